# Initial kernel scaffold; baseline (speedup 1.0000x reference)
import jax
import jax.numpy as jnp
from jax.experimental import pallas as pl


def kernel(features, edge_index, edge_types, graph_ids, W_lin, b_lin, W_ih, W_hh, b_ih, b_hh, W_c, b_c):
    raise NotImplementedError("write your pallas kernel here")


# speedup vs baseline: 8.4365x; 8.4365x over previous
import functools

import jax
import jax.numpy as jnp
from jax import lax
from jax.experimental import pallas as pl
from jax.experimental.pallas import tpu as pltpu
from jax.experimental.pallas import tpu_sc as plsc

N = 10000
E = 320000
H = 128
ETYPES = 4
STEPS = 8
B = 16

NW = 32
EPW = E // NW
CHUNK = 80
NCHUNK = EPW // CHUNK
ACC_N = 10240
RPS = ACC_N // 16
ZROWS = 128

_HI = jax.lax.Precision.HIGHEST


def _edge_body(y_hbm, src_hbm, dst_hbm, et_hbm, out_hbm,
               acc, srcb, dstb, etb, idxb, rows, zbuf, sem):
    c = lax.axis_index("c")
    s = lax.axis_index("s")
    wid = s * 2 + c

    def _zero(i, carry):
        for j in range(H // 16):
            zbuf[i, pl.ds(j * 16, 16)] = jnp.zeros((16,), jnp.float32)
        return carry
    lax.fori_loop(0, ZROWS, _zero, 0)
    for k in range(RPS // ZROWS):
        pltpu.sync_copy(zbuf, acc.at[pl.ds(s * RPS + k * ZROWS, ZROWS)])
    plsc.subcore_barrier()

    def _chunk(ci, carry):
        base = wid * EPW + ci * CHUNK
        pltpu.sync_copy(src_hbm.at[pl.ds(base, CHUNK)], srcb)
        pltpu.sync_copy(et_hbm.at[pl.ds(base, CHUNK)], etb)
        pltpu.sync_copy(dst_hbm.at[pl.ds(base, CHUNK)], dstb)
        for j in range(CHUNK // 16):
            sl = pl.ds(j * 16, 16)
            idxb[sl] = srcb[sl] * ETYPES + etb[sl]
        pltpu.async_copy(y_hbm.at[idxb], rows, sem).wait()
        pltpu.sync_copy(rows, acc.at[dstb], add=True)
        return carry
    lax.fori_loop(0, NCHUNK, _chunk, 0)
    plsc.subcore_barrier()

    for k in range(RPS // ZROWS):
        r0 = s * RPS + k * ZROWS
        pltpu.sync_copy(acc.at[pl.ds(r0, ZROWS)], out_hbm.at[c, pl.ds(r0, ZROWS)])


_edge_kernel_cache = []


def _edge_kernel(yflat, src, dst, et):
    if not _edge_kernel_cache:
        _edge_kernel_cache.append(functools.partial(
            pl.kernel,
            out_type=jax.ShapeDtypeStruct((2, ACC_N, H), jnp.float32),
            mesh=plsc.VectorSubcoreMesh(core_axis_name="c", subcore_axis_name="s",
                                        num_cores=2, num_subcores=16),
            scratch_types=[
                pltpu.VMEM_SHARED((ACC_N, H), jnp.float32),
                pltpu.VMEM((CHUNK,), jnp.int32),
                pltpu.VMEM((CHUNK,), jnp.int32),
                pltpu.VMEM((CHUNK,), jnp.int32),
                pltpu.VMEM((CHUNK,), jnp.int32),
                pltpu.VMEM((CHUNK, H), jnp.float32),
                pltpu.VMEM((ZROWS, H), jnp.float32),
                pltpu.SemaphoreType.DMA,
            ],
        )(_edge_body))
    return _edge_kernel_cache[0](yflat, src, dst, et)


ROWS_BLK = 1000
GRID = N // ROWS_BLK


def _ytc_body(x_ref, wl_ref, bl_ref, y_ref):
    y_ref[...] = (jnp.dot(x_ref[...], wl_ref[...], precision=_HI,
                          preferred_element_type=jnp.float32) + bl_ref[...])


def _gru_body(emit_y, p_ref, h_ref, wih_ref, whh_ref, bih_ref, bhh_ref,
              wl_ref, bl_ref, h_out, *maybe_y):
    a = p_ref[0] + p_ref[1]
    h = h_ref[...]
    gi = jnp.dot(a, wih_ref[...], precision=_HI,
                 preferred_element_type=jnp.float32) + bih_ref[...]
    gh = jnp.dot(h, whh_ref[...], precision=_HI,
                 preferred_element_type=jnp.float32) + bhh_ref[...]
    r = jax.nn.sigmoid(gi[:, :H] + gh[:, :H])
    z = jax.nn.sigmoid(gi[:, H:2 * H] + gh[:, H:2 * H])
    n = jnp.tanh(gi[:, 2 * H:] + r * gh[:, 2 * H:])
    hn = (1.0 - z) * n + z * h
    h_out[...] = hn
    if emit_y:
        maybe_y[0][...] = (jnp.dot(hn, wl_ref[...], precision=_HI,
                                   preferred_element_type=jnp.float32)
                           + bl_ref[...])


def _readout_body(h_ref, gid_ref, wc_ref, bc_ref, out_ref, acc):
    i = pl.program_id(0)

    @pl.when(i == 0)
    def _():
        acc[...] = jnp.zeros_like(acc)

    ids = gid_ref[0]
    iota = lax.broadcasted_iota(jnp.int32, (B, ROWS_BLK), 0)
    onehot = (iota == ids).astype(jnp.float32)
    acc[...] += lax.dot_general(onehot, h_ref[...], (((1,), (0,)), ((), ())),
                                precision=_HI, preferred_element_type=jnp.float32)

    @pl.when(i == GRID - 1)
    def _():
        logits = jnp.sum(acc[...] * wc_ref[...], axis=1) + bc_ref[0, 0]
        out_ref[...] = jax.nn.sigmoid(logits)[None, :]


def _full(shape):
    return pl.BlockSpec(shape, lambda i: (0,) * len(shape))


_y_kernel = pl.pallas_call(
    _ytc_body,
    grid=(GRID,),
    in_specs=[pl.BlockSpec((ROWS_BLK, H), lambda i: (i, 0)),
              _full((H, ETYPES * H)), _full((1, ETYPES * H))],
    out_specs=pl.BlockSpec((ROWS_BLK, ETYPES * H), lambda i: (i, 0)),
    out_shape=jax.ShapeDtypeStruct((N, ETYPES * H), jnp.float32),
)

_gru_common = dict(
    grid=(GRID,),
    in_specs=[pl.BlockSpec((2, ROWS_BLK, H), lambda i: (0, i, 0)),
              pl.BlockSpec((ROWS_BLK, H), lambda i: (i, 0)),
              _full((H, 3 * H)), _full((H, 3 * H)),
              _full((1, 3 * H)), _full((1, 3 * H)),
              _full((H, ETYPES * H)), _full((1, ETYPES * H))],
)

_gru_y_kernel = pl.pallas_call(
    functools.partial(_gru_body, True),
    out_specs=[pl.BlockSpec((ROWS_BLK, H), lambda i: (i, 0)),
               pl.BlockSpec((ROWS_BLK, ETYPES * H), lambda i: (i, 0))],
    out_shape=[jax.ShapeDtypeStruct((N, H), jnp.float32),
               jax.ShapeDtypeStruct((N, ETYPES * H), jnp.float32)],
    **_gru_common,
)

_gru_last_kernel = pl.pallas_call(
    functools.partial(_gru_body, False),
    out_specs=[pl.BlockSpec((ROWS_BLK, H), lambda i: (i, 0))],
    out_shape=[jax.ShapeDtypeStruct((N, H), jnp.float32)],
    **_gru_common,
)

_readout_kernel = pl.pallas_call(
    _readout_body,
    grid=(GRID,),
    in_specs=[pl.BlockSpec((ROWS_BLK, H), lambda i: (i, 0)),
              pl.BlockSpec((1, 1, ROWS_BLK), lambda i: (i, 0, 0)),
              _full((1, H)), _full((1, 1))],
    out_specs=pl.BlockSpec((1, B), lambda i: (0, 0)),
    out_shape=jax.ShapeDtypeStruct((1, B), jnp.float32),
    scratch_shapes=[pltpu.VMEM((B, H), jnp.float32)],
)


def kernel(features, edge_index, edge_types, graph_ids, W_lin, b_lin,
           W_ih, W_hh, b_ih, b_hh, W_c, b_c):
    wl = jnp.transpose(W_lin, (2, 0, 1)).reshape(H, ETYPES * H)
    bl = b_lin.reshape(1, ETYPES * H)
    wih = W_ih.T
    whh = W_hh.T
    bih = b_ih.reshape(1, 3 * H)
    bhh = b_hh.reshape(1, 3 * H)
    src = edge_index[0]
    dst = edge_index[1]
    gid3 = graph_ids.reshape(GRID, 1, ROWS_BLK)

    h = features
    y = _y_kernel(h, wl, bl)
    for step in range(STEPS):
        yflat = y.reshape(ETYPES * N, H)
        p = _edge_kernel(yflat, src, dst, edge_types)
        if step < STEPS - 1:
            h, y = _gru_y_kernel(p, h, wih, whh, bih, bhh, wl, bl)
        else:
            (h,) = _gru_last_kernel(p, h, wih, whh, bih, bhh, wl, bl)
    out2 = _readout_kernel(h, gid3, W_c, b_c.reshape(1, 1))
    return out2[0]


# speedup vs baseline: 15.8569x; 1.8796x over previous
import functools

import jax
import jax.numpy as jnp
from jax import lax
from jax.experimental import pallas as pl
from jax.experimental.pallas import tpu as pltpu
from jax.experimental.pallas import tpu_sc as plsc

N = 10000
E = 320000
H = 128
ETYPES = 4
STEPS = 8
B = 16

NW = 32
EPW = E // NW
CHUNK = 80
NCHUNK = EPW // CHUNK
ACC_N = 10240
RPS = ACC_N // 16
ZROWS = 128

_HI = jax.lax.Precision.HIGHEST


def _edge_body(y_hbm, gidx_hbm, dst_hbm, out_hbm,
               acc, dstb, gidxb, rows0, rows1, sem0, sem1):
    c = lax.axis_index("c")
    s = lax.axis_index("s")
    wid = s * 2 + c

    pltpu.sync_copy(gidx_hbm.at[wid], gidxb)
    pltpu.sync_copy(dst_hbm.at[wid], dstb)

    def _zero(i, carry):
        for j in range(H // 16):
            rows0[i, pl.ds(j * 16, 16)] = jnp.zeros((16,), jnp.float32)
        return carry
    lax.fori_loop(0, CHUNK, _zero, 0)
    for k in range(RPS // CHUNK):
        pltpu.sync_copy(rows0, acc.at[pl.ds(s * RPS + k * CHUNK, CHUNK)])
    plsc.subcore_barrier()

    def _gather(ci, buf, sem):
        return pltpu.async_copy(y_hbm.at[gidxb.at[pl.ds(ci * CHUNK, CHUNK)]], buf, sem)

    def _scat(ci, buf):
        pltpu.sync_copy(buf, acc.at[dstb.at[ci]], add=True)

    _gather(0, rows0, sem0)

    def _wait(ci, buf, sem):
        pltpu.make_async_copy(
            y_hbm.at[gidxb.at[pl.ds(ci * CHUNK, CHUNK)]], buf, sem).wait()

    def _pair(i, carry):
        _wait(2 * i, rows0, sem0)
        _gather(2 * i + 1, rows1, sem1)
        _scat(2 * i, rows0)
        _wait(2 * i + 1, rows1, sem1)
        _gather(2 * i + 2, rows0, sem0)
        _scat(2 * i + 1, rows1)
        return carry
    lax.fori_loop(0, (NCHUNK - 1) // 2, _pair, 0)
    _wait(NCHUNK - 1, rows0, sem0)
    _scat(NCHUNK - 1, rows0)
    plsc.subcore_barrier()

    for k in range(RPS // ZROWS):
        r0 = s * RPS + k * ZROWS
        pltpu.sync_copy(acc.at[pl.ds(r0, ZROWS)], out_hbm.at[c, pl.ds(r0, ZROWS)])


_edge_kernel_cache = []


def _edge_kernel(yflat, gidx, dst):
    if not _edge_kernel_cache:
        _edge_kernel_cache.append(functools.partial(
            pl.kernel,
            out_type=jax.ShapeDtypeStruct((2, ACC_N, H), jnp.float32),
            mesh=plsc.VectorSubcoreMesh(core_axis_name="c", subcore_axis_name="s",
                                        num_cores=2, num_subcores=16),
            scratch_types=[
                pltpu.VMEM_SHARED((ACC_N, H), jnp.float32),
                pltpu.VMEM((NCHUNK, CHUNK), jnp.int32),
                pltpu.VMEM((EPW,), jnp.int32),
                pltpu.VMEM((CHUNK, H), jnp.float32),
                pltpu.VMEM((CHUNK, H), jnp.float32),
                pltpu.SemaphoreType.DMA,
                pltpu.SemaphoreType.DMA,
            ],
        )(_edge_body))
    return _edge_kernel_cache[0](yflat, gidx, dst)


ROWS_BLK = 1000
GRID = N // ROWS_BLK


def _ytc_body(x_ref, wl_ref, bl_ref, y_ref):
    y_ref[...] = (jnp.dot(x_ref[...], wl_ref[...], precision=_HI,
                          preferred_element_type=jnp.float32) + bl_ref[...])


def _gru_body(emit_y, p_ref, h_ref, wih_ref, whh_ref, bih_ref, bhh_ref,
              wl_ref, bl_ref, h_out, *maybe_y):
    a = p_ref[0] + p_ref[1]
    h = h_ref[...]
    gi = jnp.dot(a, wih_ref[...], precision=_HI,
                 preferred_element_type=jnp.float32) + bih_ref[...]
    gh = jnp.dot(h, whh_ref[...], precision=_HI,
                 preferred_element_type=jnp.float32) + bhh_ref[...]
    r = jax.nn.sigmoid(gi[:, :H] + gh[:, :H])
    z = jax.nn.sigmoid(gi[:, H:2 * H] + gh[:, H:2 * H])
    n = jnp.tanh(gi[:, 2 * H:] + r * gh[:, 2 * H:])
    hn = (1.0 - z) * n + z * h
    h_out[...] = hn
    if emit_y:
        maybe_y[0][...] = (jnp.dot(hn, wl_ref[...], precision=_HI,
                                   preferred_element_type=jnp.float32)
                           + bl_ref[...])


def _readout_body(h_ref, gid_ref, wc_ref, bc_ref, out_ref, acc):
    i = pl.program_id(0)

    @pl.when(i == 0)
    def _():
        acc[...] = jnp.zeros_like(acc)

    ids = gid_ref[0]
    iota = lax.broadcasted_iota(jnp.int32, (B, ROWS_BLK), 0)
    onehot = (iota == ids).astype(jnp.float32)
    acc[...] += lax.dot_general(onehot, h_ref[...], (((1,), (0,)), ((), ())),
                                precision=_HI, preferred_element_type=jnp.float32)

    @pl.when(i == GRID - 1)
    def _():
        logits = jnp.sum(acc[...] * wc_ref[...], axis=1) + bc_ref[0, 0]
        out_ref[...] = jax.nn.sigmoid(logits)[None, :]


def _full(shape):
    return pl.BlockSpec(shape, lambda i: (0,) * len(shape))


_y_kernel = pl.pallas_call(
    _ytc_body,
    grid=(GRID,),
    in_specs=[pl.BlockSpec((ROWS_BLK, H), lambda i: (i, 0)),
              _full((H, ETYPES * H)), _full((1, ETYPES * H))],
    out_specs=pl.BlockSpec((ROWS_BLK, ETYPES * H), lambda i: (i, 0)),
    out_shape=jax.ShapeDtypeStruct((N, ETYPES * H), jnp.float32),
)

_gru_common = dict(
    grid=(GRID,),
    in_specs=[pl.BlockSpec((2, ROWS_BLK, H), lambda i: (0, i, 0)),
              pl.BlockSpec((ROWS_BLK, H), lambda i: (i, 0)),
              _full((H, 3 * H)), _full((H, 3 * H)),
              _full((1, 3 * H)), _full((1, 3 * H)),
              _full((H, ETYPES * H)), _full((1, ETYPES * H))],
)

_gru_y_kernel = pl.pallas_call(
    functools.partial(_gru_body, True),
    out_specs=[pl.BlockSpec((ROWS_BLK, H), lambda i: (i, 0)),
               pl.BlockSpec((ROWS_BLK, ETYPES * H), lambda i: (i, 0))],
    out_shape=[jax.ShapeDtypeStruct((N, H), jnp.float32),
               jax.ShapeDtypeStruct((N, ETYPES * H), jnp.float32)],
    **_gru_common,
)

_gru_last_kernel = pl.pallas_call(
    functools.partial(_gru_body, False),
    out_specs=[pl.BlockSpec((ROWS_BLK, H), lambda i: (i, 0))],
    out_shape=[jax.ShapeDtypeStruct((N, H), jnp.float32)],
    **_gru_common,
)

_readout_kernel = pl.pallas_call(
    _readout_body,
    grid=(GRID,),
    in_specs=[pl.BlockSpec((ROWS_BLK, H), lambda i: (i, 0)),
              pl.BlockSpec((1, 1, ROWS_BLK), lambda i: (i, 0, 0)),
              _full((1, H)), _full((1, 1))],
    out_specs=pl.BlockSpec((1, B), lambda i: (0, 0)),
    out_shape=jax.ShapeDtypeStruct((1, B), jnp.float32),
    scratch_shapes=[pltpu.VMEM((B, H), jnp.float32)],
)


def kernel(features, edge_index, edge_types, graph_ids, W_lin, b_lin,
           W_ih, W_hh, b_ih, b_hh, W_c, b_c):
    wl = jnp.transpose(W_lin, (2, 0, 1)).reshape(H, ETYPES * H)
    bl = b_lin.reshape(1, ETYPES * H)
    wih = W_ih.T
    whh = W_hh.T
    bih = b_ih.reshape(1, 3 * H)
    bhh = b_hh.reshape(1, 3 * H)
    gidx = (edge_index[0] * ETYPES + edge_types).reshape(NW, EPW)
    dst = edge_index[1].reshape(NW, NCHUNK, CHUNK)
    gid3 = graph_ids.reshape(GRID, 1, ROWS_BLK)

    h = features
    y = _y_kernel(h, wl, bl)
    for step in range(STEPS):
        yflat = y.reshape(ETYPES * N, H)
        p = _edge_kernel(yflat, gidx, dst)
        if step < STEPS - 1:
            h, y = _gru_y_kernel(p, h, wih, whh, bih, bhh, wl, bl)
        else:
            (h,) = _gru_last_kernel(p, h, wih, whh, bih, bhh, wl, bl)
    out2 = _readout_kernel(h, gid3, W_c, b_c.reshape(1, 1))
    return out2[0]


# speedup vs baseline: 15.9372x; 1.0051x over previous
import functools

import jax
import jax.numpy as jnp
from jax import lax
from jax.experimental import pallas as pl
from jax.experimental.pallas import tpu as pltpu
from jax.experimental.pallas import tpu_sc as plsc

N = 10000
E = 320000
H = 128
ETYPES = 4
STEPS = 8
B = 16

NW = 32
EPW = E // NW
CHUNK = 80
NCHUNK = EPW // CHUNK
ACC_N = 10240
RPS = ACC_N // 16
ZROWS = 128

_HI = jax.lax.Precision.HIGHEST


def _edge_body(y_hbm, gidx_hbm, dst_hbm, out_hbm,
               acc, dstb, gidxb, rows0, rows1, sem0, sem1, sem2, sem3):
    c = lax.axis_index("c")
    s = lax.axis_index("s")
    wid = s * 2 + c

    pltpu.sync_copy(gidx_hbm.at[wid], gidxb)
    pltpu.sync_copy(dst_hbm.at[wid], dstb)

    def _zero(i, carry):
        for j in range(H // 16):
            rows0[i, pl.ds(j * 16, 16)] = jnp.zeros((16,), jnp.float32)
        return carry
    lax.fori_loop(0, CHUNK, _zero, 0)
    for k in range(RPS // CHUNK):
        pltpu.sync_copy(rows0, acc.at[pl.ds(s * RPS + k * CHUNK, CHUNK)])
    plsc.subcore_barrier()

    def _gather(ci, buf, sem):
        pltpu.async_copy(y_hbm.at[gidxb.at[pl.ds(ci * CHUNK, CHUNK)]], buf, sem)

    def _gwait(ci, buf, sem):
        pltpu.make_async_copy(
            y_hbm.at[gidxb.at[pl.ds(ci * CHUNK, CHUNK)]], buf, sem).wait()

    def _scat(ci, buf, sem):
        pltpu.async_copy(buf, acc.at[dstb.at[ci]], sem, add=True)

    def _swait(ci, buf, sem):
        pltpu.make_async_copy(buf, acc.at[dstb.at[ci]], sem).wait()

    _gather(0, rows0, sem0)
    _gather(1, rows1, sem1)
    NP = (NCHUNK - 1) // 2

    def _pair(i, carry):
        _gwait(2 * i, rows0, sem0)
        _scat(2 * i, rows0, sem2)
        _gwait(2 * i + 1, rows1, sem1)
        _scat(2 * i + 1, rows1, sem3)
        _swait(2 * i, rows0, sem2)
        _gather(2 * i + 2, rows0, sem0)

        @pl.when(i < NP - 1)
        def _():
            _swait(2 * i + 1, rows1, sem3)
            _gather(2 * i + 3, rows1, sem1)
        return carry
    lax.fori_loop(0, NP, _pair, 0)
    _gwait(NCHUNK - 1, rows0, sem0)
    _scat(NCHUNK - 1, rows0, sem2)
    _swait(NCHUNK - 2, rows1, sem3)
    _swait(NCHUNK - 1, rows0, sem2)
    plsc.subcore_barrier()

    for k in range(RPS // ZROWS):
        r0 = s * RPS + k * ZROWS
        pltpu.sync_copy(acc.at[pl.ds(r0, ZROWS)], out_hbm.at[c, pl.ds(r0, ZROWS)])


_edge_kernel_cache = []


def _edge_kernel(yflat, gidx, dst):
    if not _edge_kernel_cache:
        _edge_kernel_cache.append(functools.partial(
            pl.kernel,
            out_type=jax.ShapeDtypeStruct((2, ACC_N, H), jnp.float32),
            mesh=plsc.VectorSubcoreMesh(core_axis_name="c", subcore_axis_name="s",
                                        num_cores=2, num_subcores=16),
            scratch_types=[
                pltpu.VMEM_SHARED((ACC_N, H), jnp.float32),
                pltpu.VMEM((NCHUNK, CHUNK), jnp.int32),
                pltpu.VMEM((EPW,), jnp.int32),
                pltpu.VMEM((CHUNK, H), jnp.float32),
                pltpu.VMEM((CHUNK, H), jnp.float32),
                pltpu.SemaphoreType.DMA,
                pltpu.SemaphoreType.DMA,
                pltpu.SemaphoreType.DMA,
                pltpu.SemaphoreType.DMA,
            ],
        )(_edge_body))
    return _edge_kernel_cache[0](yflat, gidx, dst)


ROWS_BLK = 1000
GRID = N // ROWS_BLK


def _ytc_body(x_ref, wl_ref, bl_ref, y_ref):
    y_ref[...] = (jnp.dot(x_ref[...], wl_ref[...], precision=_HI,
                          preferred_element_type=jnp.float32) + bl_ref[...])


def _gru_body(emit_y, p_ref, h_ref, wih_ref, whh_ref, bih_ref, bhh_ref,
              wl_ref, bl_ref, h_out, *maybe_y):
    a = p_ref[0] + p_ref[1]
    h = h_ref[...]
    gi = jnp.dot(a, wih_ref[...], precision=_HI,
                 preferred_element_type=jnp.float32) + bih_ref[...]
    gh = jnp.dot(h, whh_ref[...], precision=_HI,
                 preferred_element_type=jnp.float32) + bhh_ref[...]
    r = jax.nn.sigmoid(gi[:, :H] + gh[:, :H])
    z = jax.nn.sigmoid(gi[:, H:2 * H] + gh[:, H:2 * H])
    n = jnp.tanh(gi[:, 2 * H:] + r * gh[:, 2 * H:])
    hn = (1.0 - z) * n + z * h
    h_out[...] = hn
    if emit_y:
        maybe_y[0][...] = (jnp.dot(hn, wl_ref[...], precision=_HI,
                                   preferred_element_type=jnp.float32)
                           + bl_ref[...])


def _readout_body(h_ref, gid_ref, wc_ref, bc_ref, out_ref, acc):
    i = pl.program_id(0)

    @pl.when(i == 0)
    def _():
        acc[...] = jnp.zeros_like(acc)

    ids = gid_ref[0]
    iota = lax.broadcasted_iota(jnp.int32, (B, ROWS_BLK), 0)
    onehot = (iota == ids).astype(jnp.float32)
    acc[...] += lax.dot_general(onehot, h_ref[...], (((1,), (0,)), ((), ())),
                                precision=_HI, preferred_element_type=jnp.float32)

    @pl.when(i == GRID - 1)
    def _():
        logits = jnp.sum(acc[...] * wc_ref[...], axis=1) + bc_ref[0, 0]
        out_ref[...] = jax.nn.sigmoid(logits)[None, :]


def _full(shape):
    return pl.BlockSpec(shape, lambda i: (0,) * len(shape))


_y_kernel = pl.pallas_call(
    _ytc_body,
    grid=(GRID,),
    in_specs=[pl.BlockSpec((ROWS_BLK, H), lambda i: (i, 0)),
              _full((H, ETYPES * H)), _full((1, ETYPES * H))],
    out_specs=pl.BlockSpec((ROWS_BLK, ETYPES * H), lambda i: (i, 0)),
    out_shape=jax.ShapeDtypeStruct((N, ETYPES * H), jnp.float32),
)

_gru_common = dict(
    grid=(GRID,),
    in_specs=[pl.BlockSpec((2, ROWS_BLK, H), lambda i: (0, i, 0)),
              pl.BlockSpec((ROWS_BLK, H), lambda i: (i, 0)),
              _full((H, 3 * H)), _full((H, 3 * H)),
              _full((1, 3 * H)), _full((1, 3 * H)),
              _full((H, ETYPES * H)), _full((1, ETYPES * H))],
)

_gru_y_kernel = pl.pallas_call(
    functools.partial(_gru_body, True),
    out_specs=[pl.BlockSpec((ROWS_BLK, H), lambda i: (i, 0)),
               pl.BlockSpec((ROWS_BLK, ETYPES * H), lambda i: (i, 0))],
    out_shape=[jax.ShapeDtypeStruct((N, H), jnp.float32),
               jax.ShapeDtypeStruct((N, ETYPES * H), jnp.float32)],
    **_gru_common,
)

_gru_last_kernel = pl.pallas_call(
    functools.partial(_gru_body, False),
    out_specs=[pl.BlockSpec((ROWS_BLK, H), lambda i: (i, 0))],
    out_shape=[jax.ShapeDtypeStruct((N, H), jnp.float32)],
    **_gru_common,
)

_readout_kernel = pl.pallas_call(
    _readout_body,
    grid=(GRID,),
    in_specs=[pl.BlockSpec((ROWS_BLK, H), lambda i: (i, 0)),
              pl.BlockSpec((1, 1, ROWS_BLK), lambda i: (i, 0, 0)),
              _full((1, H)), _full((1, 1))],
    out_specs=pl.BlockSpec((1, B), lambda i: (0, 0)),
    out_shape=jax.ShapeDtypeStruct((1, B), jnp.float32),
    scratch_shapes=[pltpu.VMEM((B, H), jnp.float32)],
)


def kernel(features, edge_index, edge_types, graph_ids, W_lin, b_lin,
           W_ih, W_hh, b_ih, b_hh, W_c, b_c):
    wl = jnp.transpose(W_lin, (2, 0, 1)).reshape(H, ETYPES * H)
    bl = b_lin.reshape(1, ETYPES * H)
    wih = W_ih.T
    whh = W_hh.T
    bih = b_ih.reshape(1, 3 * H)
    bhh = b_hh.reshape(1, 3 * H)
    gidx = (edge_index[0] * ETYPES + edge_types).reshape(NW, EPW)
    dst = edge_index[1].reshape(NW, NCHUNK, CHUNK)
    gid3 = graph_ids.reshape(GRID, 1, ROWS_BLK)

    h = features
    y = _y_kernel(h, wl, bl)
    for step in range(STEPS):
        yflat = y.reshape(ETYPES * N, H)
        p = _edge_kernel(yflat, gidx, dst)
        if step < STEPS - 1:
            h, y = _gru_y_kernel(p, h, wih, whh, bih, bhh, wl, bl)
        else:
            (h,) = _gru_last_kernel(p, h, wih, whh, bih, bhh, wl, bl)
    out2 = _readout_kernel(h, gid3, W_c, b_c.reshape(1, 1))
    return out2[0]


# speedup vs baseline: 16.0172x; 1.0050x over previous
import functools

import jax
import jax.numpy as jnp
from jax import lax
from jax.experimental import pallas as pl
from jax.experimental.pallas import tpu as pltpu
from jax.experimental.pallas import tpu_sc as plsc

N = 10000
E = 320000
H = 128
ETYPES = 4
STEPS = 8
B = 16

NW = 32
EPW = E // NW
CHUNK = 80
NCHUNK = EPW // CHUNK
ACC_N = 10240
RPS = ACC_N // 16
ZROWS = 128

_HI = jax.lax.Precision.HIGHEST


def _edge_body(y_hbm, gidx_hbm, dst_hbm, out_hbm,
               acc, dstb, gidxb, rows0, rows1, sem0, sem1, sem2, sem3):
    c = lax.axis_index("c")
    s = lax.axis_index("s")
    wid = s * 2 + c

    pltpu.sync_copy(gidx_hbm.at[wid], gidxb)
    pltpu.sync_copy(dst_hbm.at[wid], dstb)

    def _zero(i, carry):
        for j in range(H // 16):
            rows0[i, pl.ds(j * 16, 16)] = jnp.zeros((16,), jnp.float32)
        return carry
    lax.fori_loop(0, CHUNK, _zero, 0)
    for k in range(RPS // CHUNK):
        pltpu.sync_copy(rows0, acc.at[pl.ds(s * RPS + k * CHUNK, CHUNK)])
    plsc.subcore_barrier()

    def _gather(ci, buf, sem):
        pltpu.async_copy(y_hbm.at[gidxb.at[pl.ds(ci * CHUNK, CHUNK)]], buf, sem)

    def _gwait(ci, buf, sem):
        pltpu.make_async_copy(
            y_hbm.at[gidxb.at[pl.ds(ci * CHUNK, CHUNK)]], buf, sem).wait()

    def _scat(ci, buf, sem):
        pltpu.async_copy(buf, acc.at[dstb.at[ci]], sem, add=True)

    def _swait(ci, buf, sem):
        pltpu.make_async_copy(buf, acc.at[dstb.at[ci]], sem).wait()

    _gather(0, rows0, sem0)
    _gather(1, rows1, sem1)
    NP = (NCHUNK - 1) // 2

    def _pair(i, carry):
        _gwait(2 * i, rows0, sem0)
        _scat(2 * i, rows0, sem2)
        _gwait(2 * i + 1, rows1, sem1)
        _scat(2 * i + 1, rows1, sem3)
        _swait(2 * i, rows0, sem2)
        _gather(2 * i + 2, rows0, sem0)

        @pl.when(i < NP - 1)
        def _():
            _swait(2 * i + 1, rows1, sem3)
            _gather(2 * i + 3, rows1, sem1)
        return carry
    lax.fori_loop(0, NP, _pair, 0)
    _gwait(NCHUNK - 1, rows0, sem0)
    _scat(NCHUNK - 1, rows0, sem2)
    _swait(NCHUNK - 2, rows1, sem3)
    _swait(NCHUNK - 1, rows0, sem2)
    plsc.subcore_barrier()

    for k in range(RPS // ZROWS):
        r0 = s * RPS + k * ZROWS
        pltpu.sync_copy(acc.at[pl.ds(r0, ZROWS)], out_hbm.at[c, pl.ds(r0, ZROWS)])


_edge_kernel_cache = []


def _edge_kernel(yflat, gidx, dst):
    if not _edge_kernel_cache:
        _edge_kernel_cache.append(functools.partial(
            pl.kernel,
            out_type=jax.ShapeDtypeStruct((2, ACC_N, H), jnp.float32),
            mesh=plsc.VectorSubcoreMesh(core_axis_name="c", subcore_axis_name="s",
                                        num_cores=2, num_subcores=16),
            scratch_types=[
                pltpu.VMEM_SHARED((ACC_N, H), jnp.float32),
                pltpu.VMEM((NCHUNK, CHUNK), jnp.int32),
                pltpu.VMEM((EPW,), jnp.int32),
                pltpu.VMEM((CHUNK, H), jnp.float32),
                pltpu.VMEM((CHUNK, H), jnp.float32),
                pltpu.SemaphoreType.DMA,
                pltpu.SemaphoreType.DMA,
                pltpu.SemaphoreType.DMA,
                pltpu.SemaphoreType.DMA,
            ],
        )(_edge_body))
    return _edge_kernel_cache[0](yflat, gidx, dst)


ROWS_BLK = 1000
GRID = N // ROWS_BLK


def _ytc_body(x_ref, wl_ref, bl_ref, y_ref):
    y_ref[...] = (jnp.dot(x_ref[...], wl_ref[...], precision=_HI,
                          preferred_element_type=jnp.float32) + bl_ref[...])


def _gru_core(p_ref, h_ref, wih_ref, whh_ref, bih_ref, bhh_ref):
    a = p_ref[0] + p_ref[1]
    h = h_ref[...]
    gi = jnp.dot(a, wih_ref[...], precision=_HI,
                 preferred_element_type=jnp.float32) + bih_ref[...]
    gh = jnp.dot(h, whh_ref[...], precision=_HI,
                 preferred_element_type=jnp.float32) + bhh_ref[...]
    r = jax.nn.sigmoid(gi[:, :H] + gh[:, :H])
    z = jax.nn.sigmoid(gi[:, H:2 * H] + gh[:, H:2 * H])
    n = jnp.tanh(gi[:, 2 * H:] + r * gh[:, 2 * H:])
    return (1.0 - z) * n + z * h


def _gru_body(p_ref, h_ref, wih_ref, whh_ref, bih_ref, bhh_ref,
              wl_ref, bl_ref, h_out, y_out):
    hn = _gru_core(p_ref, h_ref, wih_ref, whh_ref, bih_ref, bhh_ref)
    h_out[...] = hn
    y_out[...] = (jnp.dot(hn, wl_ref[...], precision=_HI,
                          preferred_element_type=jnp.float32) + bl_ref[...])


def _gru_readout_body(p_ref, h_ref, wih_ref, whh_ref, bih_ref, bhh_ref,
                      gid_ref, wc_ref, bc_ref, out_ref, acc):
    i = pl.program_id(0)
    hn = _gru_core(p_ref, h_ref, wih_ref, whh_ref, bih_ref, bhh_ref)

    @pl.when(i == 0)
    def _():
        acc[...] = jnp.zeros_like(acc)

    ids = gid_ref[0]
    iota = lax.broadcasted_iota(jnp.int32, (B, ROWS_BLK), 0)
    onehot = (iota == ids).astype(jnp.float32)
    acc[...] += lax.dot_general(onehot, hn, (((1,), (0,)), ((), ())),
                                precision=_HI, preferred_element_type=jnp.float32)

    @pl.when(i == GRID - 1)
    def _():
        logits = jnp.sum(acc[...] * wc_ref[...], axis=1) + bc_ref[0, 0]
        out_ref[...] = jax.nn.sigmoid(logits)[None, :]


def _full(shape):
    return pl.BlockSpec(shape, lambda i: (0,) * len(shape))


_y_kernel = pl.pallas_call(
    _ytc_body,
    grid=(GRID,),
    in_specs=[pl.BlockSpec((ROWS_BLK, H), lambda i: (i, 0)),
              _full((H, ETYPES * H)), _full((1, ETYPES * H))],
    out_specs=pl.BlockSpec((ROWS_BLK, ETYPES * H), lambda i: (i, 0)),
    out_shape=jax.ShapeDtypeStruct((N, ETYPES * H), jnp.float32),
)

_gru_in_specs = [pl.BlockSpec((2, ROWS_BLK, H), lambda i: (0, i, 0)),
                 pl.BlockSpec((ROWS_BLK, H), lambda i: (i, 0)),
                 _full((H, 3 * H)), _full((H, 3 * H)),
                 _full((1, 3 * H)), _full((1, 3 * H))]

_gru_y_kernel = pl.pallas_call(
    _gru_body,
    grid=(GRID,),
    in_specs=_gru_in_specs + [_full((H, ETYPES * H)), _full((1, ETYPES * H))],
    out_specs=[pl.BlockSpec((ROWS_BLK, H), lambda i: (i, 0)),
               pl.BlockSpec((ROWS_BLK, ETYPES * H), lambda i: (i, 0))],
    out_shape=[jax.ShapeDtypeStruct((N, H), jnp.float32),
               jax.ShapeDtypeStruct((N, ETYPES * H), jnp.float32)],
)

_gru_readout_kernel = pl.pallas_call(
    _gru_readout_body,
    grid=(GRID,),
    in_specs=_gru_in_specs + [pl.BlockSpec((1, 1, ROWS_BLK), lambda i: (i, 0, 0)),
                              _full((1, H)), _full((1, 1))],
    out_specs=pl.BlockSpec((1, B), lambda i: (0, 0)),
    out_shape=jax.ShapeDtypeStruct((1, B), jnp.float32),
    scratch_shapes=[pltpu.VMEM((B, H), jnp.float32)],
)


def kernel(features, edge_index, edge_types, graph_ids, W_lin, b_lin,
           W_ih, W_hh, b_ih, b_hh, W_c, b_c):
    wl = jnp.transpose(W_lin, (2, 0, 1)).reshape(H, ETYPES * H)
    bl = b_lin.reshape(1, ETYPES * H)
    wih = W_ih.T
    whh = W_hh.T
    bih = b_ih.reshape(1, 3 * H)
    bhh = b_hh.reshape(1, 3 * H)
    gidx = (edge_index[0] * ETYPES + edge_types).reshape(NW, EPW)
    dst = edge_index[1].reshape(NW, NCHUNK, CHUNK)
    gid3 = graph_ids.reshape(GRID, 1, ROWS_BLK)

    h = features
    y = _y_kernel(h, wl, bl)
    for step in range(STEPS):
        yflat = y.reshape(ETYPES * N, H)
        p = _edge_kernel(yflat, gidx, dst)
        if step < STEPS - 1:
            h, y = _gru_y_kernel(p, h, wih, whh, bih, bhh, wl, bl)
        else:
            out2 = _gru_readout_kernel(p, h, wih, whh, bih, bhh,
                                       gid3, W_c, b_c.reshape(1, 1))
    return out2[0]


# speedup vs baseline: 18.8376x; 1.1761x over previous
import functools

import jax
import jax.numpy as jnp
from jax import lax
from jax.experimental import pallas as pl
from jax.experimental.pallas import tpu as pltpu
from jax.experimental.pallas import tpu_sc as plsc

N = 10000
E = 320000
H = 128
ETYPES = 4
STEPS = 8
B = 16

NW = 32
EPW = E // NW
CHUNK = 80
NCHUNK = EPW // CHUNK
ACC_N = 10240
RPS = ACC_N // 16
ZROWS = 128

_HI = jax.lax.Precision.DEFAULT


def _edge_body(y_hbm, gidx_hbm, dst_hbm, out_hbm,
               acc, dstb, gidxb, rows0, rows1, sem0, sem1, sem2, sem3):
    c = lax.axis_index("c")
    s = lax.axis_index("s")
    wid = s * 2 + c

    pltpu.sync_copy(gidx_hbm.at[wid], gidxb)
    pltpu.sync_copy(dst_hbm.at[wid], dstb)

    def _zero(i, carry):
        for j in range(H // 16):
            rows0[i, pl.ds(j * 16, 16)] = jnp.zeros((16,), jnp.float32)
        return carry
    lax.fori_loop(0, CHUNK, _zero, 0)
    for k in range(RPS // CHUNK):
        pltpu.sync_copy(rows0, acc.at[pl.ds(s * RPS + k * CHUNK, CHUNK)])
    plsc.subcore_barrier()

    def _gather(ci, buf, sem):
        pltpu.async_copy(y_hbm.at[gidxb.at[pl.ds(ci * CHUNK, CHUNK)]], buf, sem)

    def _gwait(ci, buf, sem):
        pltpu.make_async_copy(
            y_hbm.at[gidxb.at[pl.ds(ci * CHUNK, CHUNK)]], buf, sem).wait()

    def _scat(ci, buf, sem):
        pltpu.async_copy(buf, acc.at[dstb.at[ci]], sem, add=True)

    def _swait(ci, buf, sem):
        pltpu.make_async_copy(buf, acc.at[dstb.at[ci]], sem).wait()

    _gather(0, rows0, sem0)
    _gather(1, rows1, sem1)
    NP = (NCHUNK - 1) // 2

    def _pair(i, carry):
        _gwait(2 * i, rows0, sem0)
        _scat(2 * i, rows0, sem2)
        _gwait(2 * i + 1, rows1, sem1)
        _scat(2 * i + 1, rows1, sem3)
        _swait(2 * i, rows0, sem2)
        _gather(2 * i + 2, rows0, sem0)

        @pl.when(i < NP - 1)
        def _():
            _swait(2 * i + 1, rows1, sem3)
            _gather(2 * i + 3, rows1, sem1)
        return carry
    lax.fori_loop(0, NP, _pair, 0)
    _gwait(NCHUNK - 1, rows0, sem0)
    _scat(NCHUNK - 1, rows0, sem2)
    _swait(NCHUNK - 2, rows1, sem3)
    _swait(NCHUNK - 1, rows0, sem2)
    plsc.subcore_barrier()

    for k in range(RPS // ZROWS):
        r0 = s * RPS + k * ZROWS
        pltpu.sync_copy(acc.at[pl.ds(r0, ZROWS)], out_hbm.at[c, pl.ds(r0, ZROWS)])


_edge_kernel_cache = []


def _edge_kernel(yflat, gidx, dst):
    if not _edge_kernel_cache:
        _edge_kernel_cache.append(functools.partial(
            pl.kernel,
            out_type=jax.ShapeDtypeStruct((2, ACC_N, H), jnp.float32),
            mesh=plsc.VectorSubcoreMesh(core_axis_name="c", subcore_axis_name="s",
                                        num_cores=2, num_subcores=16),
            scratch_types=[
                pltpu.VMEM_SHARED((ACC_N, H), jnp.float32),
                pltpu.VMEM((NCHUNK, CHUNK), jnp.int32),
                pltpu.VMEM((EPW,), jnp.int32),
                pltpu.VMEM((CHUNK, H), jnp.float32),
                pltpu.VMEM((CHUNK, H), jnp.float32),
                pltpu.SemaphoreType.DMA,
                pltpu.SemaphoreType.DMA,
                pltpu.SemaphoreType.DMA,
                pltpu.SemaphoreType.DMA,
            ],
        )(_edge_body))
    return _edge_kernel_cache[0](yflat, gidx, dst)


ROWS_BLK = 1000
GRID = N // ROWS_BLK


def _ytc_body(x_ref, wl_ref, bl_ref, y_ref):
    y_ref[...] = (jnp.dot(x_ref[...], wl_ref[...], precision=_HI,
                          preferred_element_type=jnp.float32) + bl_ref[...])


def _gru_core(p_ref, h_ref, wih_ref, whh_ref, bih_ref, bhh_ref):
    a = p_ref[0] + p_ref[1]
    h = h_ref[...]
    gi = jnp.dot(a, wih_ref[...], precision=_HI,
                 preferred_element_type=jnp.float32) + bih_ref[...]
    gh = jnp.dot(h, whh_ref[...], precision=_HI,
                 preferred_element_type=jnp.float32) + bhh_ref[...]
    r = jax.nn.sigmoid(gi[:, :H] + gh[:, :H])
    z = jax.nn.sigmoid(gi[:, H:2 * H] + gh[:, H:2 * H])
    n = jnp.tanh(gi[:, 2 * H:] + r * gh[:, 2 * H:])
    return (1.0 - z) * n + z * h


def _gru_body(p_ref, h_ref, wih_ref, whh_ref, bih_ref, bhh_ref,
              wl_ref, bl_ref, h_out, y_out):
    hn = _gru_core(p_ref, h_ref, wih_ref, whh_ref, bih_ref, bhh_ref)
    h_out[...] = hn
    y_out[...] = (jnp.dot(hn, wl_ref[...], precision=_HI,
                          preferred_element_type=jnp.float32) + bl_ref[...])


def _gru_readout_body(p_ref, h_ref, wih_ref, whh_ref, bih_ref, bhh_ref,
                      gid_ref, wc_ref, bc_ref, out_ref, acc):
    i = pl.program_id(0)
    hn = _gru_core(p_ref, h_ref, wih_ref, whh_ref, bih_ref, bhh_ref)

    @pl.when(i == 0)
    def _():
        acc[...] = jnp.zeros_like(acc)

    ids = gid_ref[0]
    iota = lax.broadcasted_iota(jnp.int32, (B, ROWS_BLK), 0)
    onehot = (iota == ids).astype(jnp.float32)
    acc[...] += lax.dot_general(onehot, hn, (((1,), (0,)), ((), ())),
                                precision=_HI, preferred_element_type=jnp.float32)

    @pl.when(i == GRID - 1)
    def _():
        logits = jnp.sum(acc[...] * wc_ref[...], axis=1) + bc_ref[0, 0]
        out_ref[...] = jax.nn.sigmoid(logits)[None, :]


def _full(shape):
    return pl.BlockSpec(shape, lambda i: (0,) * len(shape))


_y_kernel = pl.pallas_call(
    _ytc_body,
    grid=(GRID,),
    in_specs=[pl.BlockSpec((ROWS_BLK, H), lambda i: (i, 0)),
              _full((H, ETYPES * H)), _full((1, ETYPES * H))],
    out_specs=pl.BlockSpec((ROWS_BLK, ETYPES * H), lambda i: (i, 0)),
    out_shape=jax.ShapeDtypeStruct((N, ETYPES * H), jnp.float32),
)

_gru_in_specs = [pl.BlockSpec((2, ROWS_BLK, H), lambda i: (0, i, 0)),
                 pl.BlockSpec((ROWS_BLK, H), lambda i: (i, 0)),
                 _full((H, 3 * H)), _full((H, 3 * H)),
                 _full((1, 3 * H)), _full((1, 3 * H))]

_gru_y_kernel = pl.pallas_call(
    _gru_body,
    grid=(GRID,),
    in_specs=_gru_in_specs + [_full((H, ETYPES * H)), _full((1, ETYPES * H))],
    out_specs=[pl.BlockSpec((ROWS_BLK, H), lambda i: (i, 0)),
               pl.BlockSpec((ROWS_BLK, ETYPES * H), lambda i: (i, 0))],
    out_shape=[jax.ShapeDtypeStruct((N, H), jnp.float32),
               jax.ShapeDtypeStruct((N, ETYPES * H), jnp.float32)],
)

_gru_readout_kernel = pl.pallas_call(
    _gru_readout_body,
    grid=(GRID,),
    in_specs=_gru_in_specs + [pl.BlockSpec((1, 1, ROWS_BLK), lambda i: (i, 0, 0)),
                              _full((1, H)), _full((1, 1))],
    out_specs=pl.BlockSpec((1, B), lambda i: (0, 0)),
    out_shape=jax.ShapeDtypeStruct((1, B), jnp.float32),
    scratch_shapes=[pltpu.VMEM((B, H), jnp.float32)],
)


def kernel(features, edge_index, edge_types, graph_ids, W_lin, b_lin,
           W_ih, W_hh, b_ih, b_hh, W_c, b_c):
    wl = jnp.transpose(W_lin, (2, 0, 1)).reshape(H, ETYPES * H)
    bl = b_lin.reshape(1, ETYPES * H)
    wih = W_ih.T
    whh = W_hh.T
    bih = b_ih.reshape(1, 3 * H)
    bhh = b_hh.reshape(1, 3 * H)
    gidx = (edge_index[0] * ETYPES + edge_types).reshape(NW, EPW)
    dst = edge_index[1].reshape(NW, NCHUNK, CHUNK)
    gid3 = graph_ids.reshape(GRID, 1, ROWS_BLK)

    h = features
    y = _y_kernel(h, wl, bl)
    for step in range(STEPS):
        yflat = y.reshape(ETYPES * N, H)
        p = _edge_kernel(yflat, gidx, dst)
        if step < STEPS - 1:
            h, y = _gru_y_kernel(p, h, wih, whh, bih, bhh, wl, bl)
        else:
            out2 = _gru_readout_kernel(p, h, wih, whh, bih, bhh,
                                       gid3, W_c, b_c.reshape(1, 1))
    return out2[0]


# speedup vs baseline: 21.0569x; 1.1178x over previous
import functools

import jax
import jax.numpy as jnp
from jax import lax
from jax.experimental import pallas as pl
from jax.experimental.pallas import tpu as pltpu
from jax.experimental.pallas import tpu_sc as plsc

N = 10000
E = 320000
H = 128
ETYPES = 4
STEPS = 8
B = 16

NW = 32
EPW = E // NW
CHUNK = 80
NCHUNK = EPW // CHUNK
ACC_N = 10240
RPS = ACC_N // 16
ZROWS = 128

_HI = jax.lax.Precision.DEFAULT


def _edge_body(y_hbm, gidx_hbm, dst_hbm, out_hbm,
               acc, dstb, gidxb, rows0, rows1, sem0, sem1, sem2, sem3):
    c = lax.axis_index("c")
    s = lax.axis_index("s")
    wid = s * 2 + c

    pltpu.sync_copy(gidx_hbm.at[wid], gidxb)
    pltpu.sync_copy(dst_hbm.at[wid], dstb)

    def _zero(i, carry):
        for j in range(H // 16):
            rows0[i, pl.ds(j * 16, 16)] = jnp.zeros((16,), jnp.float32)
        return carry
    lax.fori_loop(0, CHUNK, _zero, 0)
    for k in range(RPS // CHUNK):
        pltpu.sync_copy(rows0, acc.at[pl.ds(s * RPS + k * CHUNK, CHUNK)])
    plsc.subcore_barrier()

    def _gather(ci, buf, sem):
        pltpu.async_copy(y_hbm.at[gidxb.at[pl.ds(ci * CHUNK, CHUNK)]], buf, sem)

    def _gwait(ci, buf, sem):
        pltpu.make_async_copy(
            y_hbm.at[gidxb.at[pl.ds(ci * CHUNK, CHUNK)]], buf, sem).wait()

    def _scat(ci, buf, sem):
        pltpu.async_copy(buf, acc.at[dstb.at[ci]], sem, add=True)

    def _swait(ci, buf, sem):
        pltpu.make_async_copy(buf, acc.at[dstb.at[ci]], sem).wait()

    _gather(0, rows0, sem0)
    _gather(1, rows1, sem1)
    NP = (NCHUNK - 1) // 2

    def _pair(i, carry):
        _gwait(2 * i, rows0, sem0)
        _scat(2 * i, rows0, sem2)
        _gwait(2 * i + 1, rows1, sem1)
        _scat(2 * i + 1, rows1, sem3)
        _swait(2 * i, rows0, sem2)
        _gather(2 * i + 2, rows0, sem0)

        @pl.when(i < NP - 1)
        def _():
            _swait(2 * i + 1, rows1, sem3)
            _gather(2 * i + 3, rows1, sem1)
        return carry
    lax.fori_loop(0, NP, _pair, 0)
    _gwait(NCHUNK - 1, rows0, sem0)
    _scat(NCHUNK - 1, rows0, sem2)
    _swait(NCHUNK - 2, rows1, sem3)
    _swait(NCHUNK - 1, rows0, sem2)
    plsc.subcore_barrier()

    for k in range(RPS // ZROWS):
        r0 = s * RPS + k * ZROWS
        pltpu.sync_copy(acc.at[pl.ds(r0, ZROWS)], out_hbm.at[c, pl.ds(r0, ZROWS)])


_edge_kernel_cache = []


def _edge_kernel(yflat, gidx, dst):
    if not _edge_kernel_cache:
        _edge_kernel_cache.append(functools.partial(
            pl.kernel,
            out_type=jax.ShapeDtypeStruct((2, ACC_N, H), jnp.float32),
            mesh=plsc.VectorSubcoreMesh(core_axis_name="c", subcore_axis_name="s",
                                        num_cores=2, num_subcores=16),
            scratch_types=[
                pltpu.VMEM_SHARED((ACC_N, H), jnp.float32),
                pltpu.VMEM((NCHUNK, CHUNK), jnp.int32),
                pltpu.VMEM((EPW,), jnp.int32),
                pltpu.VMEM((CHUNK, H), jnp.float32),
                pltpu.VMEM((CHUNK, H), jnp.float32),
                pltpu.SemaphoreType.DMA,
                pltpu.SemaphoreType.DMA,
                pltpu.SemaphoreType.DMA,
                pltpu.SemaphoreType.DMA,
            ],
        )(_edge_body))
    return _edge_kernel_cache[0](yflat, gidx, dst)


ROWS_BLK = 1000
GRID = N // ROWS_BLK


def _emit_y(hn, wl_ref, bl_ref, y_out):
    yc = jnp.dot(hn, wl_ref[...], precision=_HI,
                 preferred_element_type=jnp.float32) + bl_ref[...]
    for e in range(ETYPES):
        y_out[e] = yc[:, e * H:(e + 1) * H]


def _ytc_body(x_ref, wl_ref, bl_ref, y_ref):
    _emit_y(x_ref[...], wl_ref, bl_ref, y_ref)


def _gru_core(p_ref, h_ref, wih_ref, whh_ref, bih_ref, bhh_ref):
    a = p_ref[0] + p_ref[1]
    h = h_ref[...]
    gi = jnp.dot(a, wih_ref[...], precision=_HI,
                 preferred_element_type=jnp.float32) + bih_ref[...]
    gh = jnp.dot(h, whh_ref[...], precision=_HI,
                 preferred_element_type=jnp.float32) + bhh_ref[...]
    r = jax.nn.sigmoid(gi[:, :H] + gh[:, :H])
    z = jax.nn.sigmoid(gi[:, H:2 * H] + gh[:, H:2 * H])
    n = jnp.tanh(gi[:, 2 * H:] + r * gh[:, 2 * H:])
    return (1.0 - z) * n + z * h


def _gru_body(p_ref, h_ref, wih_ref, whh_ref, bih_ref, bhh_ref,
              wl_ref, bl_ref, h_out, y_out):
    hn = _gru_core(p_ref, h_ref, wih_ref, whh_ref, bih_ref, bhh_ref)
    h_out[...] = hn
    _emit_y(hn, wl_ref, bl_ref, y_out)


def _gru_readout_body(p_ref, h_ref, wih_ref, whh_ref, bih_ref, bhh_ref,
                      gid_ref, wc_ref, bc_ref, out_ref, acc):
    i = pl.program_id(0)
    hn = _gru_core(p_ref, h_ref, wih_ref, whh_ref, bih_ref, bhh_ref)

    @pl.when(i == 0)
    def _():
        acc[...] = jnp.zeros_like(acc)

    ids = gid_ref[0]
    iota = lax.broadcasted_iota(jnp.int32, (B, ROWS_BLK), 0)
    onehot = (iota == ids).astype(jnp.float32)
    acc[...] += lax.dot_general(onehot, hn, (((1,), (0,)), ((), ())),
                                precision=_HI, preferred_element_type=jnp.float32)

    @pl.when(i == GRID - 1)
    def _():
        logits = jnp.sum(acc[...] * wc_ref[...], axis=1) + bc_ref[0, 0]
        out_ref[...] = jax.nn.sigmoid(logits)[None, :]


def _full(shape):
    return pl.BlockSpec(shape, lambda i: (0,) * len(shape))


_y_kernel = pl.pallas_call(
    _ytc_body,
    grid=(GRID,),
    in_specs=[pl.BlockSpec((ROWS_BLK, H), lambda i: (i, 0)),
              _full((H, ETYPES * H)), _full((1, ETYPES * H))],
    out_specs=pl.BlockSpec((ETYPES, ROWS_BLK, H), lambda i: (0, i, 0)),
    out_shape=jax.ShapeDtypeStruct((ETYPES, N, H), jnp.float32),
)

_gru_in_specs = [pl.BlockSpec((2, ROWS_BLK, H), lambda i: (0, i, 0)),
                 pl.BlockSpec((ROWS_BLK, H), lambda i: (i, 0)),
                 _full((H, 3 * H)), _full((H, 3 * H)),
                 _full((1, 3 * H)), _full((1, 3 * H))]

_gru_y_kernel = pl.pallas_call(
    _gru_body,
    grid=(GRID,),
    in_specs=_gru_in_specs + [_full((H, ETYPES * H)), _full((1, ETYPES * H))],
    out_specs=[pl.BlockSpec((ROWS_BLK, H), lambda i: (i, 0)),
               pl.BlockSpec((ETYPES, ROWS_BLK, H), lambda i: (0, i, 0))],
    out_shape=[jax.ShapeDtypeStruct((N, H), jnp.float32),
               jax.ShapeDtypeStruct((ETYPES, N, H), jnp.float32)],
)

_gru_readout_kernel = pl.pallas_call(
    _gru_readout_body,
    grid=(GRID,),
    in_specs=_gru_in_specs + [pl.BlockSpec((1, 1, ROWS_BLK), lambda i: (i, 0, 0)),
                              _full((1, H)), _full((1, 1))],
    out_specs=pl.BlockSpec((1, B), lambda i: (0, 0)),
    out_shape=jax.ShapeDtypeStruct((1, B), jnp.float32),
    scratch_shapes=[pltpu.VMEM((B, H), jnp.float32)],
)


def kernel(features, edge_index, edge_types, graph_ids, W_lin, b_lin,
           W_ih, W_hh, b_ih, b_hh, W_c, b_c):
    wl = jnp.transpose(W_lin, (2, 0, 1)).reshape(H, ETYPES * H)
    bl = b_lin.reshape(1, ETYPES * H)
    wih = W_ih.T
    whh = W_hh.T
    bih = b_ih.reshape(1, 3 * H)
    bhh = b_hh.reshape(1, 3 * H)
    gidx = (edge_types * N + edge_index[0]).reshape(NW, EPW)
    dst = edge_index[1].reshape(NW, NCHUNK, CHUNK)
    gid3 = graph_ids.reshape(GRID, 1, ROWS_BLK)

    h = features
    y = _y_kernel(h, wl, bl)
    for step in range(STEPS):
        yflat = y.reshape(ETYPES * N, H)
        p = _edge_kernel(yflat, gidx, dst)
        if step < STEPS - 1:
            h, y = _gru_y_kernel(p, h, wih, whh, bih, bhh, wl, bl)
        else:
            out2 = _gru_readout_kernel(p, h, wih, whh, bih, bhh,
                                       gid3, W_c, b_c.reshape(1, 1))
    return out2[0]
